# baseline (device time: 215094 ns/iter reference)
import jax
import jax.numpy as jnp
from jax import lax
from jax.experimental import pallas as pl
from jax.experimental.pallas import tpu as pltpu

N_DEV = 16
HQ = 8
DH = 128
SQ = 2048
SKV = 2048
D_MODEL = 1024
CHUNK = SQ // N_DEV
SCALE = 0.08838834764831843

CLS = 768


def _class_order():
    import numpy as np
    idx = []
    for r in range(3):
        bs = [b for b in range(32) if b % 3 == r]
        bs = bs + [0] * (12 - len(bs))
        for b in bs:
            idx.extend(range(b * 64, b * 64 + 64))
    return np.asarray(idx, dtype=np.int32)


def _body(x_ref, wq_ref, k_ref, v_ref, wo_ref, out_ref,
          cbuf, sbuf, cbufL, sbufL, ctx_ref, q_ref,
          send_sems, recv_sems, credit_sem,
          send_semsL, recv_semsL, credit_semL):
    my = lax.axis_index("i")
    left = lax.rem(my + N_DEV - 1, N_DEV)
    right = lax.rem(my + 1, N_DEV)

    barrier = pltpu.get_barrier_semaphore()
    for nbr in (left, right):
        pl.semaphore_signal(barrier, inc=1, device_id=(nbr,),
                            device_id_type=pl.DeviceIdType.MESH)
    pl.semaphore_wait(barrier, 2)

    q = jnp.dot(x_ref[...], wq_ref[...],
                preferred_element_type=jnp.float32)
    q_ref[...] = (q * SCALE).astype(jnp.bfloat16)

    colv = lax.broadcasted_iota(jnp.int32, (64, CLS), 1)

    def compute_chunk(c):

        def head_body(h, carry):
            for j in range(2):
                qb = 2 * c + j
                cls = lax.rem(qb, 3)
                r = lax.rem(3 - cls, 3)
                nvalid = jnp.where(r == 2, 640, 704)
                off = r * CLS
                q64 = q_ref[pl.ds(qb * 64, 64), pl.ds(h * DH, DH)]
                s_str = lax.dot_general(
                    q64, k_ref[h, pl.ds(off, CLS), :],
                    (((1,), (1,)), ((), ())),
                    preferred_element_type=jnp.float32)
                w_str = jnp.where(colv < nvalid, jnp.exp(s_str),
                                  jnp.float32(0.0))
                flag = (cls != 0).astype(jnp.float32)
                d_off = cls * CLS + (qb // 3) * 64
                s0 = lax.dot_general(
                    q64, k_ref[h, 0:64, :], (((1,), (1,)), ((), ())),
                    preferred_element_type=jnp.float32)
                sd = lax.dot_general(
                    q64, k_ref[h, pl.ds(d_off, 64), :],
                    (((1,), (1,)), ((), ())),
                    preferred_element_type=jnp.float32)
                w0 = jnp.exp(s0) * flag
                wd = jnp.exp(sd) * flag
                inv = 1.0 / (jnp.sum(w_str, axis=1, keepdims=True)
                             + jnp.sum(w0, axis=1, keepdims=True)
                             + jnp.sum(wd, axis=1, keepdims=True))
                ch = (
                    lax.dot_general(
                        (w_str * inv).astype(jnp.bfloat16),
                        v_ref[h, pl.ds(off, CLS), :],
                        (((1,), (0,)), ((), ())),
                        preferred_element_type=jnp.float32)
                    + lax.dot_general(
                        (w0 * inv).astype(jnp.bfloat16), v_ref[h, 0:64, :],
                        (((1,), (0,)), ((), ())),
                        preferred_element_type=jnp.float32)
                    + lax.dot_general(
                        (wd * inv).astype(jnp.bfloat16),
                        v_ref[h, pl.ds(d_off, 64), :],
                        (((1,), (0,)), ((), ())),
                        preferred_element_type=jnp.float32)
                )
                ctx_ref[pl.ds(j * 64, 64), pl.ds(h * DH, DH)] = (
                    ch.astype(jnp.bfloat16))
            return carry

        lax.fori_loop(0, HQ, head_body, 0)
        out_ref[c] = jnp.dot(ctx_ref[...], wo_ref[...],
                             preferred_element_type=jnp.float32)

    compute_chunk(my)
    for s in range(N_DEV - 1):
        slot = s % 2
        send_chunk = lax.rem(my - s + 2 * N_DEV, N_DEV)
        recv_chunk = lax.rem(my - s - 1 + 2 * N_DEV, N_DEV)
        sbuf[slot] = out_ref[send_chunk].astype(jnp.bfloat16)
        if s >= 2:
            pl.semaphore_wait(credit_sem, 1)
        rdma = pltpu.make_async_remote_copy(
            src_ref=sbuf.at[slot],
            dst_ref=cbuf.at[slot],
            send_sem=send_sems.at[slot],
            recv_sem=recv_sems.at[slot],
            device_id=(right,),
            device_id_type=pl.DeviceIdType.MESH,
        )
        rdma.start()
        compute_chunk(recv_chunk)
        rdma.wait()
        out_ref[recv_chunk] = out_ref[recv_chunk] + cbuf[slot].astype(jnp.float32)
        pl.semaphore_signal(credit_sem, inc=1, device_id=(left,),
                            device_id_type=pl.DeviceIdType.MESH)

    N_R = 8
    N_L = 7
    own = lax.rem(my + 1, N_DEV)
    seed_slot = (N_DEV - 1) % 2
    sbuf[seed_slot] = out_ref[own].astype(jnp.bfloat16)
    sbufL[0] = out_ref[own].astype(jnp.bfloat16)
    for t in range(N_R):
        s = N_DEV - 1 + t
        slot = s % 2
        slotL = t % 2
        recv_chunk = lax.rem(my - t + 2 * N_DEV, N_DEV)
        recv_chunkL = lax.rem(my + 2 + t, N_DEV)
        pl.semaphore_wait(credit_sem, 1)
        rdma = pltpu.make_async_remote_copy(
            src_ref=sbuf.at[slot],
            dst_ref=cbuf.at[slot],
            send_sem=send_sems.at[slot],
            recv_sem=recv_sems.at[slot],
            device_id=(right,),
            device_id_type=pl.DeviceIdType.MESH,
        )
        rdma.start()
        if t < N_L:
            if t >= 2:
                pl.semaphore_wait(credit_semL, 1)
            rdmaL = pltpu.make_async_remote_copy(
                src_ref=sbufL.at[slotL],
                dst_ref=cbufL.at[slotL],
                send_sem=send_semsL.at[slotL],
                recv_sem=recv_semsL.at[slotL],
                device_id=(left,),
                device_id_type=pl.DeviceIdType.MESH,
            )
            rdmaL.start()
        rdma.wait()
        out_ref[recv_chunk] = cbuf[slot].astype(jnp.float32)
        if t < N_R - 1:
            sbuf[(s + 1) % 2] = cbuf[slot]
        pl.semaphore_signal(credit_sem, inc=1, device_id=(left,),
                            device_id_type=pl.DeviceIdType.MESH)
        if t < N_L:
            rdmaL.wait()
            out_ref[recv_chunkL] = cbufL[slotL].astype(jnp.float32)
            if t < N_L - 1:
                sbufL[(t + 1) % 2] = cbufL[slotL]
            pl.semaphore_signal(credit_semL, inc=1, device_id=(right,),
                                device_id_type=pl.DeviceIdType.MESH)

    pl.semaphore_wait(credit_sem, 2)
    pl.semaphore_wait(credit_semL, 2)


def kernel(x, Wq, K_ext, V_ext, Wo):
    i = lax.axis_index("i")
    K = lax.dynamic_slice_in_dim(K_ext[0], i * HQ, HQ, axis=1)
    V = lax.dynamic_slice_in_dim(V_ext[0], i * HQ, HQ, axis=1)
    order = _class_order()
    Kh = jnp.transpose(K, (1, 0, 2)).astype(jnp.bfloat16)[:, order, :]
    Vh = jnp.transpose(V, (1, 0, 2)).astype(jnp.bfloat16)[:, order, :]
    xb = x[0].astype(jnp.bfloat16)
    Wqb = Wq.astype(jnp.bfloat16)
    Wob = Wo.astype(jnp.bfloat16)

    out = pl.pallas_call(
        _body,
        out_shape=jax.ShapeDtypeStruct((N_DEV, CHUNK, D_MODEL), jnp.float32),
        in_specs=[pl.BlockSpec(memory_space=pltpu.VMEM)] * 5,
        out_specs=pl.BlockSpec(memory_space=pltpu.VMEM),
        scratch_shapes=[
            pltpu.VMEM((2, CHUNK, D_MODEL), jnp.bfloat16),
            pltpu.VMEM((2, CHUNK, D_MODEL), jnp.bfloat16),
            pltpu.VMEM((2, CHUNK, D_MODEL), jnp.bfloat16),
            pltpu.VMEM((2, CHUNK, D_MODEL), jnp.bfloat16),
            pltpu.VMEM((CHUNK, HQ * DH), jnp.bfloat16),
            pltpu.VMEM((SQ, HQ * DH), jnp.bfloat16),
            pltpu.SemaphoreType.DMA((2,)),
            pltpu.SemaphoreType.DMA((2,)),
            pltpu.SemaphoreType.REGULAR,
            pltpu.SemaphoreType.DMA((2,)),
            pltpu.SemaphoreType.DMA((2,)),
            pltpu.SemaphoreType.REGULAR,
        ],
        compiler_params=pltpu.CompilerParams(collective_id=0),
    )(xb, Wqb, Kh, Vh, Wob)
    return out.reshape(1, SQ, D_MODEL)


# device time: 185426 ns/iter; 1.1600x vs baseline; 1.1600x over previous
import jax
import jax.numpy as jnp
from jax import lax
from jax.experimental import pallas as pl
from jax.experimental.pallas import tpu as pltpu

N_DEV = 16
HQ = 8
DH = 128
SQ = 2048
SKV = 2048
D_MODEL = 1024
CHUNK = SQ // N_DEV
SCALE = 0.08838834764831843


def _body(x_ref, wq_ref, k_ref, v_ref, wo_ref, out_ref,
          cbuf, sbuf, cbufL, sbufL, ctx_ref, q_ref,
          send_sems, recv_sems, credit_sem,
          send_semsL, recv_semsL, credit_semL):
    my = lax.axis_index("i")
    left = lax.rem(my + N_DEV - 1, N_DEV)
    right = lax.rem(my + 1, N_DEV)

    barrier = pltpu.get_barrier_semaphore()
    for nbr in (left, right):
        pl.semaphore_signal(barrier, inc=1, device_id=(nbr,),
                            device_id_type=pl.DeviceIdType.MESH)
    pl.semaphore_wait(barrier, 2)

    q = jnp.dot(x_ref[...], wq_ref[...],
                preferred_element_type=jnp.float32)
    q_ref[...] = (q * SCALE).astype(jnp.bfloat16)

    cols = lax.broadcasted_iota(jnp.int32, (CHUNK, SKV), 1) // 64
    rows0 = lax.broadcasted_iota(jnp.int32, (CHUNK, SKV), 0)

    def compute_chunk(c):
        rows = (rows0 + c * CHUNK) // 64
        mask = (rows == cols) | (cols == 0) | (lax.rem(rows + cols, 3) == 0)

        def head_body(h, carry):
            qh = q_ref[pl.ds(c * CHUNK, CHUNK), pl.ds(h * DH, DH)]
            s = lax.dot_general(qh, k_ref[h], (((1,), (1,)), ((), ())),
                                preferred_element_type=jnp.float32)
            w = jnp.where(mask, jnp.exp(s), jnp.float32(0.0))
            den = jnp.sum(w, axis=1, keepdims=True)
            ch = lax.dot_general(w.astype(jnp.bfloat16), v_ref[h],
                                 (((1,), (0,)), ((), ())),
                                 preferred_element_type=jnp.float32)
            ctx_ref[:, pl.ds(h * DH, DH)] = (ch / den).astype(jnp.bfloat16)
            return carry

        lax.fori_loop(0, HQ, head_body, 0)
        out_ref[c] = jnp.dot(ctx_ref[...], wo_ref[...],
                             preferred_element_type=jnp.float32)

    compute_chunk(my)
    for s in range(N_DEV - 1):
        slot = s % 2
        send_chunk = lax.rem(my - s + 2 * N_DEV, N_DEV)
        recv_chunk = lax.rem(my - s - 1 + 2 * N_DEV, N_DEV)
        sbuf[slot] = out_ref[send_chunk].astype(jnp.bfloat16)
        if s >= 2:
            pl.semaphore_wait(credit_sem, 1)
        rdma = pltpu.make_async_remote_copy(
            src_ref=sbuf.at[slot],
            dst_ref=cbuf.at[slot],
            send_sem=send_sems.at[slot],
            recv_sem=recv_sems.at[slot],
            device_id=(right,),
            device_id_type=pl.DeviceIdType.MESH,
        )
        rdma.start()
        compute_chunk(recv_chunk)
        rdma.wait()
        out_ref[recv_chunk] = out_ref[recv_chunk] + cbuf[slot].astype(jnp.float32)
        pl.semaphore_signal(credit_sem, inc=1, device_id=(left,),
                            device_id_type=pl.DeviceIdType.MESH)

    N_R = 8
    N_L = 7
    own = lax.rem(my + 1, N_DEV)
    seed_slot = (N_DEV - 1) % 2
    sbuf[seed_slot] = out_ref[own].astype(jnp.bfloat16)
    sbufL[0] = out_ref[own].astype(jnp.bfloat16)
    for t in range(N_R):
        s = N_DEV - 1 + t
        slot = s % 2
        slotL = t % 2
        recv_chunk = lax.rem(my - t + 2 * N_DEV, N_DEV)
        recv_chunkL = lax.rem(my + 2 + t, N_DEV)
        pl.semaphore_wait(credit_sem, 1)
        rdma = pltpu.make_async_remote_copy(
            src_ref=sbuf.at[slot],
            dst_ref=cbuf.at[slot],
            send_sem=send_sems.at[slot],
            recv_sem=recv_sems.at[slot],
            device_id=(right,),
            device_id_type=pl.DeviceIdType.MESH,
        )
        rdma.start()
        if t < N_L:
            if t >= 2:
                pl.semaphore_wait(credit_semL, 1)
            rdmaL = pltpu.make_async_remote_copy(
                src_ref=sbufL.at[slotL],
                dst_ref=cbufL.at[slotL],
                send_sem=send_semsL.at[slotL],
                recv_sem=recv_semsL.at[slotL],
                device_id=(left,),
                device_id_type=pl.DeviceIdType.MESH,
            )
            rdmaL.start()
        rdma.wait()
        out_ref[recv_chunk] = cbuf[slot].astype(jnp.float32)
        if t < N_R - 1:
            sbuf[(s + 1) % 2] = cbuf[slot]
        pl.semaphore_signal(credit_sem, inc=1, device_id=(left,),
                            device_id_type=pl.DeviceIdType.MESH)
        if t < N_L:
            rdmaL.wait()
            out_ref[recv_chunkL] = cbufL[slotL].astype(jnp.float32)
            if t < N_L - 1:
                sbufL[(t + 1) % 2] = cbufL[slotL]
            pl.semaphore_signal(credit_semL, inc=1, device_id=(right,),
                                device_id_type=pl.DeviceIdType.MESH)

    pl.semaphore_wait(credit_sem, 2)
    pl.semaphore_wait(credit_semL, 2)


def kernel(x, Wq, K_ext, V_ext, Wo):
    i = lax.axis_index("i")
    K = lax.dynamic_slice_in_dim(K_ext[0], i * HQ, HQ, axis=1)
    V = lax.dynamic_slice_in_dim(V_ext[0], i * HQ, HQ, axis=1)
    Kh = jnp.transpose(K, (1, 0, 2)).astype(jnp.bfloat16)
    Vh = jnp.transpose(V, (1, 0, 2)).astype(jnp.bfloat16)
    xb = x[0].astype(jnp.bfloat16)
    Wqb = Wq.astype(jnp.bfloat16)
    Wob = Wo.astype(jnp.bfloat16)

    out = pl.pallas_call(
        _body,
        out_shape=jax.ShapeDtypeStruct((N_DEV, CHUNK, D_MODEL), jnp.float32),
        in_specs=[pl.BlockSpec(memory_space=pltpu.VMEM)] * 5,
        out_specs=pl.BlockSpec(memory_space=pltpu.VMEM),
        scratch_shapes=[
            pltpu.VMEM((2, CHUNK, D_MODEL), jnp.bfloat16),
            pltpu.VMEM((2, CHUNK, D_MODEL), jnp.bfloat16),
            pltpu.VMEM((2, CHUNK, D_MODEL), jnp.bfloat16),
            pltpu.VMEM((2, CHUNK, D_MODEL), jnp.bfloat16),
            pltpu.VMEM((CHUNK, HQ * DH), jnp.bfloat16),
            pltpu.VMEM((SQ, HQ * DH), jnp.bfloat16),
            pltpu.SemaphoreType.DMA((2,)),
            pltpu.SemaphoreType.DMA((2,)),
            pltpu.SemaphoreType.REGULAR,
            pltpu.SemaphoreType.DMA((2,)),
            pltpu.SemaphoreType.DMA((2,)),
            pltpu.SemaphoreType.REGULAR,
        ],
        compiler_params=pltpu.CompilerParams(collective_id=0),
    )(xb, Wqb, Kh, Vh, Wob)
    return out.reshape(1, SQ, D_MODEL)
